# Initial kernel scaffold; baseline (speedup 1.0000x reference)
import functools

import jax
import jax.numpy as jnp
from jax import lax
from jax.experimental import pallas as pl
from jax.experimental.pallas import tpu as pltpu
from jax.experimental.pallas import tpu_sc as plsc

N = 10000
E = 320000
D = 128

NC = 2
NS = 16
NW = NC * NS
EPT = E // NW
K = 125
G = EPT // K
NB = 10240
ZB = NB // NS
RPS = N // NS

_mesh = plsc.VectorSubcoreMesh(core_axis_name="c", subcore_axis_name="s")


@functools.partial(
    pl.kernel,
    mesh=_mesh,
    out_type=jax.ShapeDtypeStruct((NC, 2, NB), jnp.float32),
    scratch_types=[
        pltpu.VMEM((G, K), jnp.int32),
        pltpu.VMEM((G, K), jnp.int32),
        pltpu.VMEM((128,), jnp.float32),
        pltpu.VMEM((ZB,), jnp.float32),
        pltpu.VMEM_SHARED((NB,), jnp.float32),
        pltpu.VMEM_SHARED((NB,), jnp.float32),
        pltpu.SemaphoreType.DMA,
    ],
)
def _hist_kernel(edges_hbm, out_hbm, src_v, dst_v, ones_v, buf_v,
                 csrc_sh, cdst_sh, sem):
    cid = lax.axis_index("c")
    sid = lax.axis_index("s")
    w = sid * NC + cid
    pltpu.sync_copy(edges_hbm.at[0, w], src_v)
    pltpu.sync_copy(edges_hbm.at[1, w], dst_v)

    def _fill_ones(i, _):
        ones_v[pl.ds(i * 16, 16)] = jnp.ones((16,), jnp.float32)
        return 0
    lax.fori_loop(0, 8, _fill_ones, 0)

    def _fill_zeros(i, _):
        buf_v[pl.ds(i * 16, 16)] = jnp.zeros((16,), jnp.float32)
        return 0
    lax.fori_loop(0, ZB // 16, _fill_zeros, 0)

    pltpu.sync_copy(buf_v, csrc_sh.at[pl.ds(sid * ZB, ZB)])
    pltpu.sync_copy(buf_v, cdst_sh.at[pl.ds(sid * ZB, ZB)])
    plsc.subcore_barrier()

    def _scatter(g, _):
        pltpu.sync_copy(ones_v.at[pl.ds(0, K)], csrc_sh.at[src_v.at[g]],
                        add=True)
        pltpu.sync_copy(ones_v.at[pl.ds(0, K)], cdst_sh.at[dst_v.at[g]],
                        add=True)
        return 0
    lax.fori_loop(0, G, _scatter, 0)
    plsc.subcore_barrier()

    pltpu.sync_copy(csrc_sh.at[pl.ds(sid * ZB, ZB)], buf_v)
    pltpu.sync_copy(buf_v, out_hbm.at[cid, 0, pl.ds(sid * ZB, ZB)])
    pltpu.sync_copy(cdst_sh.at[pl.ds(sid * ZB, ZB)], buf_v)
    pltpu.sync_copy(buf_v, out_hbm.at[cid, 1, pl.ds(sid * ZB, ZB)])


@functools.partial(
    pl.kernel,
    mesh=_mesh,
    out_type=jax.ShapeDtypeStruct((NC, N, D), jnp.float32),
    scratch_types=[
        pltpu.VMEM((G, K), jnp.int32),
        pltpu.VMEM((G, K), jnp.int32),
        pltpu.VMEM((K, D), jnp.float32),
        pltpu.VMEM((25, D), jnp.float32),
        pltpu.VMEM_SHARED((N, D), jnp.float32),
        pltpu.SemaphoreType.DMA,
    ],
)
def _agg_kernel(feat_hbm, edges_hbm, out_hbm, src_v, dst_v, rows_v, zb_v,
                agg_sh, sem):
    cid = lax.axis_index("c")
    sid = lax.axis_index("s")
    w = sid * NC + cid
    pltpu.sync_copy(edges_hbm.at[0, w], src_v)
    pltpu.sync_copy(edges_hbm.at[1, w], dst_v)

    def _fill_zeros(i, _):
        zb_v[i // 8, pl.ds((i % 8) * 16, 16)] = jnp.zeros((16,), jnp.float32)
        return 0
    lax.fori_loop(0, 25 * 8, _fill_zeros, 0)

    def _zero_agg(j, _):
        pltpu.sync_copy(zb_v, agg_sh.at[pl.ds(sid * RPS + j * 25, 25)])
        return 0
    lax.fori_loop(0, RPS // 25, _zero_agg, 0)
    plsc.subcore_barrier()

    def _edge_chunk(g, _):
        pltpu.async_copy(feat_hbm.at[src_v.at[g]], rows_v, sem).wait()
        pltpu.sync_copy(rows_v, agg_sh.at[dst_v.at[g]], add=True)
        return 0
    lax.fori_loop(0, G, _edge_chunk, 0)
    plsc.subcore_barrier()

    def _readback(j, _):
        base = sid * RPS + j * K
        pltpu.sync_copy(agg_sh.at[pl.ds(base, K)], rows_v)
        pltpu.sync_copy(rows_v, out_hbm.at[cid, pl.ds(base, K)])
        return 0
    lax.fori_loop(0, RPS // K, _readback, 0)


def _scale_body(deg_ref, feat_ref, out_ref):
    deg = deg_ref[0, 0] + deg_ref[1, 0]
    scale = lax.rsqrt(jnp.maximum(deg, 1.0))
    out_ref[...] = feat_ref[...] * scale


def _final_body(aggp_ref, deg_ref, wmu_ref, wlog_ref, epsw_ref,
                bmu_ref, blog_ref, epsb_ref, out_ref, kl_ref):
    wlog = wlog_ref[...]
    wmu = wmu_ref[...]
    weight = wmu + jnp.exp(wlog) * epsw_ref[...]
    agg = aggp_ref[0] + aggp_ref[1]
    rst = jnp.dot(agg, weight, preferred_element_type=jnp.float32,
                  precision=lax.Precision.HIGHEST)
    deg = deg_ref[0, 1] + deg_ref[1, 1]
    scale = lax.rsqrt(jnp.maximum(deg, 1.0))
    blog = blog_ref[...]
    bmu = bmu_ref[...]
    bias = bmu + jnp.exp(blog) * epsb_ref[...]
    out_ref[...] = rst * scale + bias

    @pl.when(pl.program_id(0) == 0)
    def _():
        klw = jnp.sum(-wlog + (jnp.exp(2.0 * wlog) + wmu * wmu) * 0.5 - 0.5)
        klb = jnp.sum(-blog + (jnp.exp(2.0 * blog) + bmu * bmu) * 0.5 - 0.5)
        kl_ref[0, 0] = klw + klb


def kernel(feat, weight_mu, weight_logsd, bias_mu, bias_logsd, edge_index):
    feat = feat.astype(jnp.float32)
    edges = edge_index.astype(jnp.int32).reshape(2, NW, G, K)

    hist = _hist_kernel(edges)
    deg_col = hist[:, :, :N].reshape(NC, 2, N, 1)

    rb = 2000
    feat_scaled = pl.pallas_call(
        _scale_body,
        grid=(N // rb,),
        in_specs=[
            pl.BlockSpec((NC, 2, rb, 1), lambda i: (0, 0, i, 0)),
            pl.BlockSpec((rb, D), lambda i: (i, 0)),
        ],
        out_specs=pl.BlockSpec((rb, D), lambda i: (i, 0)),
        out_shape=jax.ShapeDtypeStruct((N, D), jnp.float32),
    )(deg_col, feat)

    aggp = _agg_kernel(feat_scaled, edges)

    eps_w = jax.random.normal(jax.random.key(42), weight_mu.shape,
                              dtype=weight_mu.dtype)
    eps_b = jax.random.normal(jax.random.key(43), bias_mu.shape,
                              dtype=bias_mu.dtype)

    rst, kl = pl.pallas_call(
        _final_body,
        grid=(N // rb,),
        in_specs=[
            pl.BlockSpec((NC, rb, D), lambda i: (0, i, 0)),
            pl.BlockSpec((NC, 2, rb, 1), lambda i: (0, 0, i, 0)),
            pl.BlockSpec((D, D), lambda i: (0, 0)),
            pl.BlockSpec((D, D), lambda i: (0, 0)),
            pl.BlockSpec((D, D), lambda i: (0, 0)),
            pl.BlockSpec((1, D), lambda i: (0, 0)),
            pl.BlockSpec((1, D), lambda i: (0, 0)),
            pl.BlockSpec((1, D), lambda i: (0, 0)),
        ],
        out_specs=[
            pl.BlockSpec((rb, D), lambda i: (i, 0)),
            pl.BlockSpec((1, 1), lambda i: (0, 0)),
        ],
        out_shape=[
            jax.ShapeDtypeStruct((N, D), jnp.float32),
            jax.ShapeDtypeStruct((1, 1), jnp.float32),
        ],
    )(aggp, deg_col, weight_mu, weight_logsd, eps_w,
      bias_mu, bias_logsd, eps_b)

    return rst, kl[0, 0]


# speedup vs baseline: 7.5184x; 7.5184x over previous
import functools

import jax
import jax.numpy as jnp
from jax import lax
from jax.experimental import pallas as pl
from jax.experimental.pallas import tpu as pltpu
from jax.experimental.pallas import tpu_sc as plsc

N = 10000
E = 320000
D = 128

NC = 2
NS = 16
NW = NC * NS
EPT = E // NW
K = 125
G = EPT // K
NB = 10240
ZB = NB // NS
NP = 10240
RPS = NP // NS

_mesh = plsc.VectorSubcoreMesh(core_axis_name="c", subcore_axis_name="s")


@functools.partial(
    pl.kernel,
    mesh=_mesh,
    out_type=jax.ShapeDtypeStruct((NC, 2, NB), jnp.float32),
    scratch_types=[
        pltpu.VMEM((G, K), jnp.int32),
        pltpu.VMEM((G, K), jnp.int32),
        pltpu.VMEM((128,), jnp.float32),
        pltpu.VMEM((ZB,), jnp.float32),
        pltpu.VMEM_SHARED((NB,), jnp.float32),
        pltpu.VMEM_SHARED((NB,), jnp.float32),
        pltpu.SemaphoreType.DMA,
    ],
)
def _hist_kernel(edges_hbm, out_hbm, src_v, dst_v, ones_v, buf_v,
                 csrc_sh, cdst_sh, sem):
    cid = lax.axis_index("c")
    sid = lax.axis_index("s")
    w = sid * NC + cid
    pltpu.sync_copy(edges_hbm.at[0, w], src_v)
    pltpu.sync_copy(edges_hbm.at[1, w], dst_v)

    def _fill_ones(i, _):
        ones_v[pl.ds(i * 16, 16)] = jnp.ones((16,), jnp.float32)
        return 0
    lax.fori_loop(0, 8, _fill_ones, 0)

    def _fill_zeros(i, _):
        buf_v[pl.ds(i * 16, 16)] = jnp.zeros((16,), jnp.float32)
        return 0
    lax.fori_loop(0, ZB // 16, _fill_zeros, 0)

    pltpu.sync_copy(buf_v, csrc_sh.at[pl.ds(sid * ZB, ZB)])
    pltpu.sync_copy(buf_v, cdst_sh.at[pl.ds(sid * ZB, ZB)])
    plsc.subcore_barrier()

    def _scatter(g, _):
        pltpu.sync_copy(ones_v.at[pl.ds(0, K)], csrc_sh.at[src_v.at[g]],
                        add=True)
        pltpu.sync_copy(ones_v.at[pl.ds(0, K)], cdst_sh.at[dst_v.at[g]],
                        add=True)
        return 0
    lax.fori_loop(0, G, _scatter, 0)
    plsc.subcore_barrier()

    pltpu.sync_copy(csrc_sh.at[pl.ds(sid * ZB, ZB)], buf_v)
    pltpu.sync_copy(buf_v, out_hbm.at[cid, 0, pl.ds(sid * ZB, ZB)])
    pltpu.sync_copy(cdst_sh.at[pl.ds(sid * ZB, ZB)], buf_v)
    pltpu.sync_copy(buf_v, out_hbm.at[cid, 1, pl.ds(sid * ZB, ZB)])


@functools.partial(
    pl.kernel,
    mesh=_mesh,
    out_type=jax.ShapeDtypeStruct((NC, NP, D), jnp.float32),
    scratch_types=[
        pltpu.VMEM((G, K), jnp.int32),
        pltpu.VMEM((G, K), jnp.int32),
        pltpu.VMEM((128, D), jnp.float32),
        pltpu.VMEM_SHARED((NP, D), jnp.float32),
        pltpu.SemaphoreType.DMA,
    ],
)
def _agg_kernel(feat_hbm, edges_hbm, out_hbm, src_v, dst_v, rows_v,
                agg_sh, sem):
    cid = lax.axis_index("c")
    sid = lax.axis_index("s")
    w = sid * NC + cid
    pltpu.sync_copy(edges_hbm.at[0, w], src_v)
    pltpu.sync_copy(edges_hbm.at[1, w], dst_v)

    def _fill_zeros(i, _):
        rows_v[i // 8, pl.ds((i % 8) * 16, 16)] = jnp.zeros((16,), jnp.float32)
        return 0
    lax.fori_loop(0, 128 * 8, _fill_zeros, 0)

    def _zero_agg(j, _):
        pltpu.sync_copy(rows_v, agg_sh.at[pl.ds(sid * RPS + j * 128, 128)])
        return 0
    lax.fori_loop(0, RPS // 128, _zero_agg, 0)
    plsc.subcore_barrier()

    def _edge_chunk(g, _):
        pltpu.async_copy(feat_hbm.at[src_v.at[g]], rows_v.at[pl.ds(0, K)],
                         sem).wait()
        pltpu.sync_copy(rows_v.at[pl.ds(0, K)], agg_sh.at[dst_v.at[g]],
                        add=True)
        return 0
    lax.fori_loop(0, G, _edge_chunk, 0)
    plsc.subcore_barrier()

    def _readback(j, _):
        base = sid * RPS + j * 128
        pltpu.sync_copy(agg_sh.at[pl.ds(base, 128)], rows_v)
        pltpu.sync_copy(rows_v, out_hbm.at[cid, pl.ds(base, 128)])
        return 0
    lax.fori_loop(0, RPS // 128, _readback, 0)


def _scale_body(deg_ref, feat_ref, out_ref):
    deg = deg_ref[0, 0] + deg_ref[1, 0]
    scale = lax.rsqrt(jnp.maximum(deg, 1.0))
    out_ref[...] = feat_ref[...] * scale


def _final_body(aggp_ref, deg_ref, wmu_ref, wlog_ref, epsw_ref,
                bmu_ref, blog_ref, epsb_ref, out_ref, kl_ref):
    wlog = wlog_ref[...]
    wmu = wmu_ref[...]
    weight = wmu + jnp.exp(wlog) * epsw_ref[...]
    agg = aggp_ref[0] + aggp_ref[1]
    rst = jnp.dot(agg, weight, preferred_element_type=jnp.float32,
                  precision=lax.Precision.HIGHEST)
    deg = deg_ref[0, 1] + deg_ref[1, 1]
    scale = lax.rsqrt(jnp.maximum(deg, 1.0))
    blog = blog_ref[...]
    bmu = bmu_ref[...]
    bias = bmu + jnp.exp(blog) * epsb_ref[...]
    out_ref[...] = rst * scale + bias

    @pl.when(pl.program_id(0) == 0)
    def _():
        klw = jnp.sum(-wlog + (jnp.exp(2.0 * wlog) + wmu * wmu) * 0.5 - 0.5)
        klb = jnp.sum(-blog + (jnp.exp(2.0 * blog) + bmu * bmu) * 0.5 - 0.5)
        kl_ref[...] = jnp.reshape(klw + klb, (1, 1))


def kernel(feat, weight_mu, weight_logsd, bias_mu, bias_logsd, edge_index):
    feat = feat.astype(jnp.float32)
    edges = edge_index.astype(jnp.int32).reshape(2, NW, G, K)

    hist = _hist_kernel(edges)
    deg_col = hist.reshape(NC, 2, NB, 1)

    rb = 2000
    feat_scaled = pl.pallas_call(
        _scale_body,
        grid=(N // rb,),
        in_specs=[
            pl.BlockSpec((NC, 2, rb, 1), lambda i: (0, 0, i, 0)),
            pl.BlockSpec((rb, D), lambda i: (i, 0)),
        ],
        out_specs=pl.BlockSpec((rb, D), lambda i: (i, 0)),
        out_shape=jax.ShapeDtypeStruct((N, D), jnp.float32),
    )(deg_col, feat)

    aggp = _agg_kernel(feat_scaled, edges)

    eps_w = jax.random.normal(jax.random.key(42), weight_mu.shape,
                              dtype=weight_mu.dtype)
    eps_b = jax.random.normal(jax.random.key(43), bias_mu.shape,
                              dtype=bias_mu.dtype)

    rst, kl = pl.pallas_call(
        _final_body,
        grid=(N // rb,),
        in_specs=[
            pl.BlockSpec((NC, rb, D), lambda i: (0, i, 0)),
            pl.BlockSpec((NC, 2, rb, 1), lambda i: (0, 0, i, 0)),
            pl.BlockSpec((D, D), lambda i: (0, 0)),
            pl.BlockSpec((D, D), lambda i: (0, 0)),
            pl.BlockSpec((D, D), lambda i: (0, 0)),
            pl.BlockSpec((1, D), lambda i: (0, 0)),
            pl.BlockSpec((1, D), lambda i: (0, 0)),
            pl.BlockSpec((1, D), lambda i: (0, 0)),
        ],
        out_specs=[
            pl.BlockSpec((rb, D), lambda i: (i, 0)),
            pl.BlockSpec((1, 1), lambda i: (0, 0)),
        ],
        out_shape=[
            jax.ShapeDtypeStruct((N, D), jnp.float32),
            jax.ShapeDtypeStruct((1, 1), jnp.float32),
        ],
    )(aggp, deg_col, weight_mu, weight_logsd, eps_w,
      bias_mu, bias_logsd, eps_b)

    return rst, kl[0, 0]
